# baseline (speedup 1.0000x reference)
import functools

import jax
import jax.numpy as jnp
from jax import lax
from jax.experimental import pallas as pl
from jax.experimental.pallas import tpu as pltpu
from jax.experimental.pallas import tpu_sc as plsc

NUM_CORES = 2
NUM_SUBCORES = 16
NUM_WORKERS = NUM_CORES * NUM_SUBCORES

CH = 512
NBUF = 2
OBW = 4 * 8 * 128


def _make_gather(S, T, V, D):
    B = S * T
    k_per_w = (B // CH) // NUM_WORKERS
    b_per_w = k_per_w * CH
    chunks_per_t = S // CH
    t_block = D * S
    dt_block = 8 * S
    assert b_per_w * NUM_WORKERS == B and CH * chunks_per_t == S
    assert k_per_w % NBUF == 0 and k_per_w >= 3 * NBUF
    n_steps = k_per_w // NBUF

    mesh = plsc.VectorSubcoreMesh(core_axis_name="c", subcore_axis_name="s")

    scratch = (
        [pltpu.VMEM((b_per_w,), jnp.int32)]
        + [pltpu.VMEM((CH, D), jnp.float32) for _ in range(NBUF)]
        + [pltpu.VMEM((OBW,), jnp.float32) for _ in range(NBUF * 4)]
        + [pltpu.SemaphoreType.DMA for _ in range(2 * NBUF)]
    )

    @functools.partial(
        pl.kernel,
        out_type=jax.ShapeDtypeStruct((B * D,), jnp.float32),
        mesh=mesh,
        scratch_types=scratch,
        compiler_params=pltpu.CompilerParams(
            use_tc_tiling_on_sc=False, needs_layout_passes=False
        ),
    )
    def gather_kernel(idx_hbm, table_hbm, out_hbm, idx_v, *bufs):
        rows = bufs[:NBUF]
        obuf = [bufs[NBUF + 4 * b : NBUF + 4 * (b + 1)] for b in range(NBUF)]
        gsem = bufs[5 * NBUF : 6 * NBUF]
        ssem = bufs[6 * NBUF :]
        wid = lax.axis_index("s") * NUM_CORES + lax.axis_index("c")
        k0 = wid * k_per_w
        pltpu.sync_copy(idx_hbm.at[pl.ds(k0 * CH, b_per_w)], idx_v)

        iotav = lax.iota(jnp.int32, 16)
        dsplat = [jnp.full((16,), d, jnp.int32) for d in range(D)]

        def start_gather(c, b):
            pltpu.async_copy(
                table_hbm.at[idx_v.at[pl.ds(c * CH, CH)]], rows[b], gsem[b]
            )

        def wait_gather(b):
            pltpu.make_async_copy(
                table_hbm.at[idx_v.at[pl.ds(0, CH)]], rows[b], gsem[b]
            ).wait()

        def chunk_base(c):
            k = k0 + c
            t = k // chunks_per_t
            st0 = k % chunks_per_t
            return t * t_block + st0 * CH

        def start_store(c, b):
            base = chunk_base(c)
            for dt in range(4):
                pltpu.async_copy(
                    obuf[b][dt],
                    out_hbm.at[pl.ds(base + dt * dt_block, OBW)],
                    ssem[b],
                )

        def wait_store(b):
            for dt in range(4):
                pltpu.make_async_copy(
                    obuf[b][dt],
                    out_hbm.at[pl.ds(0, OBW)],
                    ssem[b],
                ).wait()

        def shuffle(b):
            rv = rows[b]
            obs = obuf[b]
            dorder = [(i % 4) * 8 + i // 4 for i in range(32)]

            def qbody(q, carry):
                jvec = q * 16 + iotav
                qoff = (q // 8) * 1024 + (q % 8) * 16
                for d in dorder:
                    v = plsc.load_gather(rv, [jvec, dsplat[d]])
                    obs[d // 8][pl.ds(qoff + (d % 8) * 128, 16)] = v
                return carry

            lax.fori_loop(0, 32, qbody, 0)

        for b in range(NBUF):
            start_gather(b, b)

        def step_body(step, carry):
            for b in range(NBUF):
                c = step * NBUF + b
                wait_gather(b)
                shuffle(b)

                @pl.when(step > 0)
                def _():
                    wait_store(b)

                start_store(c, b)
                start_gather(c + NBUF, b)
            return carry

        lax.fori_loop(0, n_steps - 1, step_body, 0)

        for b in range(NBUF):
            c = (n_steps - 1) * NBUF + b
            wait_gather(b)
            shuffle(b)
            wait_store(b)
            start_store(c, b)
        for b in range(NBUF):
            wait_store(b)

    return gather_kernel


def kernel(phonemes, table):
    S, T = phonemes.shape
    V, D = table.shape
    idx_flat = jnp.transpose(phonemes).reshape(-1).astype(jnp.int32)
    out_flat = _make_gather(S, T, V, D)(idx_flat, table)
    out5 = out_flat.reshape(T, D // 8, S // 128, 8, 128)
    x = out5.transpose(0, 1, 3, 2, 4).reshape(T, D, S)
    return x.transpose(2, 0, 1)


# speedup vs baseline: 1.2569x; 1.2569x over previous
import functools

import jax
import jax.numpy as jnp
from jax import lax
from jax.experimental import pallas as pl
from jax.experimental.pallas import tpu as pltpu
from jax.experimental.pallas import tpu_sc as plsc

NUM_CORES = 2
NUM_SUBCORES = 16
NUM_WORKERS = NUM_CORES * NUM_SUBCORES

CH = 640
NBUF = 4
BK = 8192


def _tc_repack(V, D):
    grid = pl.cdiv(V, BK)

    def body(tT_ref, out_ref):
        x = tT_ref[...]
        y = jnp.swapaxes(x, 0, 1)
        y32 = y.reshape(BK // 4, 4, D)
        out_ref[...] = jnp.concatenate(
            [y32[:, jm, :] for jm in range(4)], axis=1
        )

    return pl.pallas_call(
        body,
        grid=(grid,),
        in_specs=[pl.BlockSpec((D, BK), lambda g: (0, g))],
        out_specs=pl.BlockSpec((BK * D // 128, 128), lambda g: (g, 0)),
        out_shape=jax.ShapeDtypeStruct((V * D // 128, 128), jnp.float32),
    )


def _make_gather(S, T, V, D):
    B = S * T
    k_per_w = (B // CH) // NUM_WORKERS
    b_per_w = k_per_w * CH
    assert b_per_w * NUM_WORKERS == B
    assert k_per_w % NBUF == 0 and k_per_w >= 3 * NBUF
    n_steps = k_per_w // NBUF

    mesh = plsc.VectorSubcoreMesh(core_axis_name="c", subcore_axis_name="s")

    scratch = (
        [pltpu.VMEM((b_per_w,), jnp.int32)]
        + [pltpu.VMEM((CH, D), jnp.float32) for _ in range(NBUF)]
        + [pltpu.SemaphoreType.DMA for _ in range(2 * NBUF)]
    )

    @functools.partial(
        pl.kernel,
        out_type=jax.ShapeDtypeStruct((B, D), jnp.float32),
        mesh=mesh,
        scratch_types=scratch,
        compiler_params=pltpu.CompilerParams(
            use_tc_tiling_on_sc=False, needs_layout_passes=False
        ),
    )
    def gather_kernel(idx_hbm, table_hbm, out_hbm, idx_v, *bufs):
        rows = bufs[:NBUF]
        gsem = bufs[NBUF : 2 * NBUF]
        ssem = bufs[2 * NBUF :]
        wid = lax.axis_index("s") * NUM_CORES + lax.axis_index("c")
        j0 = wid * b_per_w
        pltpu.sync_copy(idx_hbm.at[pl.ds(j0, b_per_w)], idx_v)

        def start_gather(c, b):
            pltpu.async_copy(
                table_hbm.at[idx_v.at[pl.ds(c * CH, CH)]], rows[b], gsem[b]
            )

        def wait_gather(b):
            pltpu.make_async_copy(
                table_hbm.at[idx_v.at[pl.ds(0, CH)]], rows[b], gsem[b]
            ).wait()

        def start_store(c, b):
            pltpu.async_copy(
                rows[b],
                out_hbm.at[pl.ds(j0 + c * CH, CH)],
                ssem[b],
            )

        def wait_store(b):
            pltpu.make_async_copy(
                rows[b],
                out_hbm.at[pl.ds(0, CH)],
                ssem[b],
            ).wait()

        for b in range(NBUF):
            start_gather(b, b)

        def step_body(step, carry):
            for b in range(NBUF):
                c = step * NBUF + b
                wait_gather(b)

                @pl.when(step > 0)
                def _():
                    wait_store(b)

                start_store(c, b)
                start_gather(c + NBUF, b)
            return carry

        lax.fori_loop(0, n_steps - 1, step_body, 0)

        for b in range(NBUF):
            c = (n_steps - 1) * NBUF + b
            wait_gather(b)
            wait_store(b)
            start_store(c, b)
        for b in range(NBUF):
            wait_store(b)

    return gather_kernel


def kernel(phonemes, table):
    S, T = phonemes.shape
    V, D = table.shape
    tableT = jnp.transpose(table)
    scr = _tc_repack(V, D)(tableT)
    table_lin = scr.reshape(-1).reshape(V, D)
    idx_flat = phonemes.reshape(-1).astype(jnp.int32)
    out2 = _make_gather(S, T, V, D)(idx_flat, table_lin)
    return out2.reshape(S, T, D)


# speedup vs baseline: 1.2572x; 1.0002x over previous
import functools

import jax
import jax.numpy as jnp
from jax import lax
from jax.experimental import pallas as pl
from jax.experimental.pallas import tpu as pltpu
from jax.experimental.pallas import tpu_sc as plsc

NUM_CORES = 2
NUM_SUBCORES = 16
NUM_WORKERS = NUM_CORES * NUM_SUBCORES

CH = 640
NBUF = 4
BK = 8192


def _tc_repack(V, D):
    grid = pl.cdiv(V, BK)

    def body(tT_ref, out_ref):
        x = tT_ref[...]
        y = jnp.swapaxes(x, 0, 1)
        y32 = y.reshape(BK // 4, 4, D)
        out_ref[...] = jnp.concatenate(
            [y32[:, jm, :] for jm in range(4)], axis=1
        )

    return pl.pallas_call(
        body,
        grid=(grid,),
        in_specs=[pl.BlockSpec((D, BK), lambda g: (0, g))],
        out_specs=pl.BlockSpec((BK * D // 128, 128), lambda g: (g, 0)),
        out_shape=jax.ShapeDtypeStruct((V * D // 128, 128), jnp.float32),
    )


def _make_gather(S, T, V, D):
    B = S * T
    k_per_w = (B // CH) // NUM_WORKERS
    b_per_w = k_per_w * CH
    assert b_per_w * NUM_WORKERS == B
    assert k_per_w % NBUF == 0 and k_per_w >= 3 * NBUF
    n_steps = k_per_w // NBUF

    mesh = plsc.VectorSubcoreMesh(core_axis_name="c", subcore_axis_name="s")

    scratch = (
        [pltpu.VMEM((b_per_w,), jnp.int32)]
        + [pltpu.VMEM((CH, D), jnp.float32) for _ in range(NBUF)]
        + [pltpu.SemaphoreType.DMA for _ in range(2 * NBUF)]
    )

    @functools.partial(
        pl.kernel,
        out_type=jax.ShapeDtypeStruct((B, D), jnp.float32),
        mesh=mesh,
        scratch_types=scratch,
        compiler_params=pltpu.CompilerParams(
            use_tc_tiling_on_sc=False, needs_layout_passes=False
        ),
    )
    def gather_kernel(idx_hbm, table_hbm, out_hbm, idx_v, *bufs):
        rows = bufs[:NBUF]
        gsem = bufs[NBUF : 2 * NBUF]
        ssem = bufs[2 * NBUF :]
        wid = lax.axis_index("s") * NUM_CORES + lax.axis_index("c")
        j0 = wid * b_per_w
        pltpu.sync_copy(idx_hbm.at[pl.ds(j0, b_per_w)], idx_v)

        def start_gather(c, b):
            pltpu.async_copy(
                table_hbm.at[idx_v.at[pl.ds(c * CH, CH)]], rows[b], gsem[b]
            )

        def wait_gather(b):
            pltpu.make_async_copy(
                table_hbm.at[idx_v.at[pl.ds(0, CH)]], rows[b], gsem[b]
            ).wait()

        def start_store(c, b):
            pltpu.async_copy(
                rows[b],
                out_hbm.at[pl.ds(j0 + c * CH, CH)],
                ssem[b],
            )

        def wait_store(b):
            pltpu.make_async_copy(
                rows[b],
                out_hbm.at[pl.ds(0, CH)],
                ssem[b],
            ).wait()

        for b in range(NBUF):
            start_gather(b, b)

        def step_body(step, carry):
            for b in range(NBUF):
                c = step * NBUF + b
                wait_gather(b)
                start_store(c, b)
                wait_store(b)
                start_gather(c + NBUF, b)
            return carry

        lax.fori_loop(0, n_steps - 1, step_body, 0)

        for b in range(NBUF):
            c = (n_steps - 1) * NBUF + b
            wait_gather(b)
            start_store(c, b)
        for b in range(NBUF):
            wait_store(b)

    return gather_kernel


def kernel(phonemes, table):
    S, T = phonemes.shape
    V, D = table.shape
    tableT = jnp.transpose(table)
    scr = _tc_repack(V, D)(tableT)
    table_lin = scr.reshape(-1).reshape(V, D)
    idx_flat = phonemes.reshape(-1).astype(jnp.int32)
    out2 = _make_gather(S, T, V, D)(idx_flat, table_lin)
    return out2.reshape(S, T, D)
